# baseline (speedup 1.0000x reference)
import functools

import jax
import jax.numpy as jnp
from jax import lax
from jax.experimental import pallas as pl
from jax.experimental.pallas import tpu as pltpu
from jax.experimental.pallas import tpu_sc as plsc

_EMB = 256
_K = 1024
_M = 512


def _dist_argmin_body(xf_ref, w2_ref, b2_ref, idx_ref):
    xf = xf_ref[...]
    a2 = jnp.sum(xf * xf, axis=1, keepdims=True)
    mm2 = lax.dot_general(xf, w2_ref[...], (((1,), (1,)), ((), ())),
                          preferred_element_type=jnp.float32)
    dist = jnp.sqrt(jnp.maximum((a2 + b2_ref[...]) - mm2, 0.0))
    mv = jnp.min(dist, axis=1, keepdims=True)
    ks = lax.broadcasted_iota(jnp.int32, dist.shape, 1)
    idx = jnp.min(jnp.where(dist == mv, ks, _K), axis=1)
    idx_ref[...] = idx[None, None, :].astype(jnp.int32)


def _tc_indices(xf, W2, b2):
    n = xf.shape[0]
    nb = n // _M
    return pl.pallas_call(
        _dist_argmin_body,
        grid=(nb,),
        in_specs=[
            pl.BlockSpec((_M, _EMB), lambda i: (i, 0)),
            pl.BlockSpec((_K, _EMB), lambda i: (0, 0)),
            pl.BlockSpec((1, _K), lambda i: (0, 0)),
        ],
        out_specs=pl.BlockSpec((1, 1, _M), lambda i: (i, 0, 0)),
        out_shape=jax.ShapeDtypeStruct((nb, 1, _M), jnp.int32),
        compiler_params=pltpu.CompilerParams(
            dimension_semantics=("arbitrary",)),
    )(xf, W2, b2)


_NC, _NS = 2, 16
_NW = _NC * _NS
_CH = 128


def _make_sc_gather(n):
    _BPW = n // _NW
    _NCHUNK = _BPW // _CH
    @functools.partial(
        pl.kernel,
        mesh=plsc.VectorSubcoreMesh(core_axis_name="c", subcore_axis_name="s"),
        out_type=jax.ShapeDtypeStruct((n, _EMB), jnp.float32),
        scratch_types=[
            pltpu.VMEM((_CH,), jnp.int32),
            pltpu.VMEM((_CH,), jnp.int32),
            pltpu.VMEM((_CH, _EMB), jnp.float32),
            pltpu.VMEM((_CH, _EMB), jnp.float32),
            pltpu.SemaphoreType.DMA,
            pltpu.SemaphoreType.DMA,
            pltpu.SemaphoreType.DMA,
            pltpu.SemaphoreType.DMA,
        ],
    )
    def _sc_gather(table_hbm, idx_hbm, out_hbm, idx_v0, idx_v1, rows_v0,
                   rows_v1, sg0, sg1, sw0, sw1):
        wid = lax.axis_index("s") * _NC + lax.axis_index("c")
        base = wid * _BPW
        idx_v = (idx_v0, idx_v1)
        rows_v = (rows_v0, rows_v1)
        sg = (sg0, sg1)
        sw = (sw0, sw1)
        gathers = [None] * _NCHUNK
        writes = [None] * _NCHUNK
        pltpu.sync_copy(idx_hbm.at[pl.ds(base, _CH)], idx_v0)
        gathers[0] = pltpu.async_copy(table_hbm.at[idx_v0], rows_v0, sg0)
        for ci in range(_NCHUNK):
            p = ci % 2
            if ci + 1 < _NCHUNK:
                q = (ci + 1) % 2
                pltpu.sync_copy(
                    idx_hbm.at[pl.ds(base + (ci + 1) * _CH, _CH)], idx_v[q])
                if ci >= 1:
                    writes[ci - 1].wait()
                gathers[ci + 1] = pltpu.async_copy(
                    table_hbm.at[idx_v[q]], rows_v[q], sg[q])
            gathers[ci].wait()
            writes[ci] = pltpu.async_copy(
                rows_v[p], out_hbm.at[pl.ds(base + ci * _CH, _CH)], sw[p])
        writes[_NCHUNK - 2].wait()
        writes[_NCHUNK - 1].wait()

    return _sc_gather


def kernel(x, W):
    b, c, h, w = x.shape
    n = b * h * w
    xf = jnp.transpose(x, (0, 2, 3, 1)).reshape(n, c)
    b2 = jnp.sum(W * W, axis=1)[None, :]
    W2 = 2.0 * W
    half = n // 2
    sc_gather = _make_sc_gather(half)
    idx0 = _tc_indices(xf[:half], W2, b2).reshape(half)
    idx1 = _tc_indices(xf[half:], W2, b2).reshape(half)
    qf0 = sc_gather(W, idx0)
    qf1 = sc_gather(W, idx1)
    idx = jnp.concatenate([idx0, idx1])
    qf = jnp.concatenate([qf0, qf1])
    quantized = jnp.transpose(qf.reshape(b, h, w, c), (0, 3, 1, 2))
    return (quantized, idx)


# speedup vs baseline: 1.0839x; 1.0839x over previous
import functools

import jax
import jax.numpy as jnp
from jax import lax
from jax.experimental import pallas as pl
from jax.experimental.pallas import tpu as pltpu
from jax.experimental.pallas import tpu_sc as plsc

_EMB = 256
_K = 1024
_M = 512


def _dist_argmin_body(xf_ref, w2_ref, b2_ref, idx_ref):
    xf = xf_ref[...]
    a2 = jnp.sum(xf * xf, axis=1, keepdims=True)
    mm2 = lax.dot_general(xf, w2_ref[...], (((1,), (1,)), ((), ())),
                          preferred_element_type=jnp.float32)
    d2c = jnp.maximum((a2 + b2_ref[...]) - mm2, 0.0)
    mv2 = jnp.min(d2c, axis=1, keepdims=True)
    sm = mv2.reshape(_M // 128, 128)
    smin = jnp.sqrt(sm)
    t0 = lax.bitcast_convert_type(smin * smin, jnp.int32)
    tbest = jnp.zeros_like(t0)
    for j in range(-3, 6):
        xj_bits = jnp.maximum(t0 + j, 0)
        xj = lax.bitcast_convert_type(xj_bits, jnp.float32)
        ok = jnp.sqrt(xj) <= smin
        tbest = jnp.where(ok, xj_bits, tbest)
    thr = lax.bitcast_convert_type(tbest, jnp.float32).reshape(_M, 1)
    ks = lax.broadcasted_iota(jnp.int32, d2c.shape, 1)
    idx = jnp.min(jnp.where(d2c <= thr, ks, _K), axis=1)
    idx_ref[...] = idx[None, None, :].astype(jnp.int32)


def _tc_indices(xf, W2, b2):
    n = xf.shape[0]
    nb = n // _M
    return pl.pallas_call(
        _dist_argmin_body,
        grid=(nb,),
        in_specs=[
            pl.BlockSpec((_M, _EMB), lambda i: (i, 0)),
            pl.BlockSpec((_K, _EMB), lambda i: (0, 0)),
            pl.BlockSpec((1, _K), lambda i: (0, 0)),
        ],
        out_specs=pl.BlockSpec((1, 1, _M), lambda i: (i, 0, 0)),
        out_shape=jax.ShapeDtypeStruct((nb, 1, _M), jnp.int32),
        compiler_params=pltpu.CompilerParams(
            dimension_semantics=("arbitrary",)),
    )(xf, W2, b2)


_NC, _NS = 2, 16
_NW = _NC * _NS
_CH = 128


def _make_sc_gather(n):
    _BPW = n // _NW
    _NCHUNK = _BPW // _CH
    @functools.partial(
        pl.kernel,
        mesh=plsc.VectorSubcoreMesh(core_axis_name="c", subcore_axis_name="s"),
        out_type=jax.ShapeDtypeStruct((n, _EMB), jnp.float32),
        scratch_types=[
            pltpu.VMEM((_CH,), jnp.int32),
            pltpu.VMEM((_CH,), jnp.int32),
            pltpu.VMEM((_CH, _EMB), jnp.float32),
            pltpu.VMEM((_CH, _EMB), jnp.float32),
            pltpu.SemaphoreType.DMA,
            pltpu.SemaphoreType.DMA,
            pltpu.SemaphoreType.DMA,
            pltpu.SemaphoreType.DMA,
        ],
    )
    def _sc_gather(table_hbm, idx_hbm, out_hbm, idx_v0, idx_v1, rows_v0,
                   rows_v1, sg0, sg1, sw0, sw1):
        wid = lax.axis_index("s") * _NC + lax.axis_index("c")
        base = wid * _BPW
        idx_v = (idx_v0, idx_v1)
        rows_v = (rows_v0, rows_v1)
        sg = (sg0, sg1)
        sw = (sw0, sw1)
        gathers = [None] * _NCHUNK
        writes = [None] * _NCHUNK
        pltpu.sync_copy(idx_hbm.at[pl.ds(base, _CH)], idx_v0)
        gathers[0] = pltpu.async_copy(table_hbm.at[idx_v0], rows_v0, sg0)
        for ci in range(_NCHUNK):
            p = ci % 2
            if ci + 1 < _NCHUNK:
                q = (ci + 1) % 2
                pltpu.sync_copy(
                    idx_hbm.at[pl.ds(base + (ci + 1) * _CH, _CH)], idx_v[q])
                if ci >= 1:
                    writes[ci - 1].wait()
                gathers[ci + 1] = pltpu.async_copy(
                    table_hbm.at[idx_v[q]], rows_v[q], sg[q])
            gathers[ci].wait()
            writes[ci] = pltpu.async_copy(
                rows_v[p], out_hbm.at[pl.ds(base + ci * _CH, _CH)], sw[p])
        writes[_NCHUNK - 2].wait()
        writes[_NCHUNK - 1].wait()

    return _sc_gather


def kernel(x, W):
    b, c, h, w = x.shape
    n = b * h * w
    xf = jnp.transpose(x, (0, 2, 3, 1)).reshape(n, c)
    b2 = jnp.sum(W * W, axis=1)[None, :]
    idx = _tc_indices(xf, 2.0 * W, b2).reshape(n)
    qf = _make_sc_gather(n)(W, idx)
    quantized = jnp.transpose(qf.reshape(b, h, w, c), (0, 3, 1, 2))
    return (quantized, idx)


# speedup vs baseline: 1.0935x; 1.0088x over previous
import functools

import jax
import jax.numpy as jnp
from jax import lax
from jax.experimental import pallas as pl
from jax.experimental.pallas import tpu as pltpu
from jax.experimental.pallas import tpu_sc as plsc

_EMB = 256
_K = 1024
_M = 512


def _dist_argmin_body(xb_ref, w2_ref, b2_ref, a2_ref, idx_ref):
    xb = xb_ref[0]
    a2 = a2_ref[0].reshape(-1, 1)
    mm2 = lax.dot_general(xb, w2_ref[...], (((0,), (1,)), ((), ())),
                          preferred_element_type=jnp.float32)
    dist = jnp.sqrt(jnp.maximum((a2 + b2_ref[...]) - mm2, 0.0))
    mv = jnp.min(dist, axis=1, keepdims=True)
    ks = lax.broadcasted_iota(jnp.int32, dist.shape, 1)
    idx = jnp.min(jnp.where(dist == mv, ks, _K), axis=1)
    idx_ref[...] = idx[None, None, :].astype(jnp.int32)


def _tc_indices(x3, W2, b2, a2):
    nb, c, hw = x3.shape
    return pl.pallas_call(
        _dist_argmin_body,
        grid=(nb,),
        in_specs=[
            pl.BlockSpec((1, c, hw), lambda i: (i, 0, 0)),
            pl.BlockSpec((_K, _EMB), lambda i: (0, 0)),
            pl.BlockSpec((1, _K), lambda i: (0, 0)),
            pl.BlockSpec((1, 1, hw), lambda i: (i, 0, 0)),
        ],
        out_specs=pl.BlockSpec((1, 1, hw), lambda i: (i, 0, 0)),
        out_shape=jax.ShapeDtypeStruct((nb, 1, hw), jnp.int32),
        compiler_params=pltpu.CompilerParams(
            dimension_semantics=("arbitrary",)),
    )(x3, W2, b2, a2)


_NC, _NS = 2, 16
_NW = _NC * _NS
_CH = 128


def _make_sc_gather(n):
    _BPW = n // _NW
    _NCHUNK = _BPW // _CH
    @functools.partial(
        pl.kernel,
        mesh=plsc.VectorSubcoreMesh(core_axis_name="c", subcore_axis_name="s"),
        out_type=jax.ShapeDtypeStruct((n, _EMB), jnp.float32),
        scratch_types=[
            pltpu.VMEM((_CH,), jnp.int32),
            pltpu.VMEM((_CH,), jnp.int32),
            pltpu.VMEM((_CH, _EMB), jnp.float32),
            pltpu.VMEM((_CH, _EMB), jnp.float32),
            pltpu.SemaphoreType.DMA,
            pltpu.SemaphoreType.DMA,
            pltpu.SemaphoreType.DMA,
            pltpu.SemaphoreType.DMA,
        ],
    )
    def _sc_gather(table_hbm, idx_hbm, out_hbm, idx_v0, idx_v1, rows_v0,
                   rows_v1, sg0, sg1, sw0, sw1):
        wid = lax.axis_index("s") * _NC + lax.axis_index("c")
        base = wid * _BPW
        idx_v = (idx_v0, idx_v1)
        rows_v = (rows_v0, rows_v1)
        sg = (sg0, sg1)
        sw = (sw0, sw1)
        gathers = [None] * _NCHUNK
        writes = [None] * _NCHUNK
        pltpu.sync_copy(idx_hbm.at[pl.ds(base, _CH)], idx_v0)
        gathers[0] = pltpu.async_copy(table_hbm.at[idx_v0], rows_v0, sg0)
        for ci in range(_NCHUNK):
            p = ci % 2
            if ci + 1 < _NCHUNK:
                q = (ci + 1) % 2
                pltpu.sync_copy(
                    idx_hbm.at[pl.ds(base + (ci + 1) * _CH, _CH)], idx_v[q])
                if ci >= 1:
                    writes[ci - 1].wait()
                gathers[ci + 1] = pltpu.async_copy(
                    table_hbm.at[idx_v[q]], rows_v[q], sg[q])
            gathers[ci].wait()
            writes[ci] = pltpu.async_copy(
                rows_v[p], out_hbm.at[pl.ds(base + ci * _CH, _CH)], sw[p])
        writes[_NCHUNK - 2].wait()
        writes[_NCHUNK - 1].wait()

    return _sc_gather


def kernel(x, W):
    b, c, h, w = x.shape
    n = b * h * w
    b2 = jnp.sum(W * W, axis=1)[None, :]
    a2 = jnp.sum(x * x, axis=1).reshape(b, 1, h * w)
    idx = _tc_indices(x.reshape(b, c, h * w), 2.0 * W, b2, a2).reshape(n)
    qf = _make_sc_gather(n)(W, idx)
    quantized = jnp.transpose(qf.reshape(b, h, w, c), (0, 3, 1, 2))
    return (quantized, idx)


# speedup vs baseline: 1.2222x; 1.1177x over previous
import functools

import jax
import jax.numpy as jnp
from jax import lax
from jax.experimental import pallas as pl
from jax.experimental.pallas import tpu as pltpu
from jax.experimental.pallas import tpu_sc as plsc

_EMB = 256
_K = 1024
_M = 512


def _dist_argmin_body(xb_ref, w2_ref, b2_ref, idx_ref):
    xb = xb_ref[0]
    xf = xb.T
    a2 = jnp.sum(xf * xf, axis=1, keepdims=True)
    mm2 = lax.dot_general(xb, w2_ref[...], (((0,), (1,)), ((), ())),
                          preferred_element_type=jnp.float32)
    dist = jnp.sqrt(jnp.maximum((a2 + b2_ref[...]) - mm2, 0.0))
    mv = jnp.min(dist, axis=1, keepdims=True)
    ks = lax.broadcasted_iota(jnp.int32, dist.shape, 1)
    idx = jnp.min(jnp.where(dist == mv, ks, _K), axis=1)
    idx_ref[...] = idx[None, None, :].astype(jnp.int32)


def _tc_indices(x3, W2, b2):
    nb, c, hw = x3.shape
    return pl.pallas_call(
        _dist_argmin_body,
        grid=(nb,),
        in_specs=[
            pl.BlockSpec((1, c, hw), lambda i: (i, 0, 0)),
            pl.BlockSpec((_K, _EMB), lambda i: (0, 0)),
            pl.BlockSpec((1, _K), lambda i: (0, 0)),
        ],
        out_specs=pl.BlockSpec((1, 1, hw), lambda i: (i, 0, 0)),
        out_shape=jax.ShapeDtypeStruct((nb, 1, hw), jnp.int32),
        compiler_params=pltpu.CompilerParams(
            dimension_semantics=("arbitrary",)),
    )(x3, W2, b2)


_NC, _NS = 2, 16
_NW = _NC * _NS
_CH = 128


def _make_sc_gather(n):
    _BPW = n // _NW
    _NCHUNK = _BPW // _CH
    @functools.partial(
        pl.kernel,
        mesh=plsc.VectorSubcoreMesh(core_axis_name="c", subcore_axis_name="s"),
        out_type=jax.ShapeDtypeStruct((n, _EMB), jnp.float32),
        scratch_types=[
            pltpu.VMEM((_CH,), jnp.int32),
            pltpu.VMEM((_CH,), jnp.int32),
            pltpu.VMEM((_CH, _EMB), jnp.float32),
            pltpu.VMEM((_CH, _EMB), jnp.float32),
            pltpu.SemaphoreType.DMA,
            pltpu.SemaphoreType.DMA,
            pltpu.SemaphoreType.DMA,
            pltpu.SemaphoreType.DMA,
        ],
    )
    def _sc_gather(table_hbm, idx_hbm, out_hbm, idx_v0, idx_v1, rows_v0,
                   rows_v1, sg0, sg1, sw0, sw1):
        wid = lax.axis_index("s") * _NC + lax.axis_index("c")
        base = wid * _BPW
        idx_v = (idx_v0, idx_v1)
        rows_v = (rows_v0, rows_v1)
        sg = (sg0, sg1)
        sw = (sw0, sw1)
        gathers = [None] * _NCHUNK
        writes = [None] * _NCHUNK
        pltpu.sync_copy(idx_hbm.at[pl.ds(base, _CH)], idx_v0)
        gathers[0] = pltpu.async_copy(table_hbm.at[idx_v0], rows_v0, sg0)
        for ci in range(_NCHUNK):
            p = ci % 2
            if ci + 1 < _NCHUNK:
                q = (ci + 1) % 2
                pltpu.sync_copy(
                    idx_hbm.at[pl.ds(base + (ci + 1) * _CH, _CH)], idx_v[q])
                if ci >= 1:
                    writes[ci - 1].wait()
                gathers[ci + 1] = pltpu.async_copy(
                    table_hbm.at[idx_v[q]], rows_v[q], sg[q])
            gathers[ci].wait()
            writes[ci] = pltpu.async_copy(
                rows_v[p], out_hbm.at[pl.ds(base + ci * _CH, _CH)], sw[p])
        writes[_NCHUNK - 2].wait()
        writes[_NCHUNK - 1].wait()

    return _sc_gather


def kernel(x, W):
    b, c, h, w = x.shape
    n = b * h * w
    b2 = jnp.sum(W * W, axis=1)[None, :]
    idx = _tc_indices(x.reshape(b, c, h * w), 2.0 * W, b2).reshape(n)
    qf = _make_sc_gather(n)(W, idx)
    quantized = jnp.transpose(qf.reshape(b, h, w, c), (0, 3, 1, 2))
    return (quantized, idx)


# speedup vs baseline: 1.3174x; 1.0779x over previous
import functools

import jax
import jax.numpy as jnp
from jax import lax
from jax.experimental import pallas as pl
from jax.experimental.pallas import tpu as pltpu
from jax.experimental.pallas import tpu_sc as plsc

_EMB = 256
_K = 1024
_M = 1024


def _dist_argmin_body(xf_ref, w2_ref, b2_ref, idx_ref):
    xf = xf_ref[...]
    a2 = jnp.sum(xf * xf, axis=1, keepdims=True)
    mm2 = lax.dot_general(xf, w2_ref[...], (((1,), (1,)), ((), ())),
                          preferred_element_type=jnp.float32)
    dist = jnp.sqrt(jnp.maximum((a2 + b2_ref[...]) - mm2, 0.0))
    mv = jnp.min(dist, axis=1, keepdims=True)
    ks = lax.broadcasted_iota(jnp.int32, dist.shape, 1)
    idx = jnp.min(jnp.where(dist == mv, ks, _K), axis=1)
    idx_ref[...] = idx[None, None, :].astype(jnp.int32)


def _tc_indices(xf, W2, b2):
    n = xf.shape[0]
    nb = n // _M
    return pl.pallas_call(
        _dist_argmin_body,
        grid=(nb,),
        in_specs=[
            pl.BlockSpec((_M, _EMB), lambda i: (i, 0)),
            pl.BlockSpec((_K, _EMB), lambda i: (0, 0)),
            pl.BlockSpec((1, _K), lambda i: (0, 0)),
        ],
        out_specs=pl.BlockSpec((1, 1, _M), lambda i: (i, 0, 0)),
        out_shape=jax.ShapeDtypeStruct((nb, 1, _M), jnp.int32),
        compiler_params=pltpu.CompilerParams(
            dimension_semantics=("arbitrary",)),
    )(xf, W2, b2)


_NC, _NS = 2, 16
_NW = _NC * _NS
_CH = 128


def _make_sc_gather(n):
    _BPW = n // _NW
    _NCHUNK = _BPW // _CH
    @functools.partial(
        pl.kernel,
        mesh=plsc.VectorSubcoreMesh(core_axis_name="c", subcore_axis_name="s"),
        out_type=jax.ShapeDtypeStruct((n, _EMB), jnp.float32),
        scratch_types=[
            pltpu.VMEM((_CH,), jnp.int32),
            pltpu.VMEM((_CH,), jnp.int32),
            pltpu.VMEM((_CH, _EMB), jnp.float32),
            pltpu.VMEM((_CH, _EMB), jnp.float32),
            pltpu.SemaphoreType.DMA,
            pltpu.SemaphoreType.DMA,
            pltpu.SemaphoreType.DMA,
            pltpu.SemaphoreType.DMA,
        ],
    )
    def _sc_gather(table_hbm, idx_hbm, out_hbm, idx_v0, idx_v1, rows_v0,
                   rows_v1, sg0, sg1, sw0, sw1):
        wid = lax.axis_index("s") * _NC + lax.axis_index("c")
        base = wid * _BPW
        idx_v = (idx_v0, idx_v1)
        rows_v = (rows_v0, rows_v1)
        sg = (sg0, sg1)
        sw = (sw0, sw1)
        gathers = [None] * _NCHUNK
        writes = [None] * _NCHUNK
        pltpu.sync_copy(idx_hbm.at[pl.ds(base, _CH)], idx_v0)
        gathers[0] = pltpu.async_copy(table_hbm.at[idx_v0], rows_v0, sg0)
        for ci in range(_NCHUNK):
            p = ci % 2
            if ci + 1 < _NCHUNK:
                q = (ci + 1) % 2
                pltpu.sync_copy(
                    idx_hbm.at[pl.ds(base + (ci + 1) * _CH, _CH)], idx_v[q])
                if ci >= 1:
                    writes[ci - 1].wait()
                gathers[ci + 1] = pltpu.async_copy(
                    table_hbm.at[idx_v[q]], rows_v[q], sg[q])
            gathers[ci].wait()
            writes[ci] = pltpu.async_copy(
                rows_v[p], out_hbm.at[pl.ds(base + ci * _CH, _CH)], sw[p])
        writes[_NCHUNK - 2].wait()
        writes[_NCHUNK - 1].wait()

    return _sc_gather


def kernel(x, W):
    b, c, h, w = x.shape
    n = b * h * w
    xf = jnp.transpose(x, (0, 2, 3, 1)).reshape(n, c)
    b2 = jnp.sum(W * W, axis=1)[None, :]
    idx = _tc_indices(xf, 2.0 * W, b2).reshape(n)
    qf = _make_sc_gather(n)(W, idx)
    quantized = jnp.transpose(qf.reshape(b, h, w, c), (0, 3, 1, 2))
    return (quantized, idx)


# speedup vs baseline: 1.4277x; 1.0838x over previous
import functools

import jax
import jax.numpy as jnp
from jax import lax
from jax.experimental import pallas as pl
from jax.experimental.pallas import tpu as pltpu
from jax.experimental.pallas import tpu_sc as plsc

_EMB = 256
_K = 1024
_M = 2048


def _dist_argmin_body(xf_ref, w2_ref, b2_ref, idx_ref):
    xf = xf_ref[...]
    a2 = jnp.sum(xf * xf, axis=1, keepdims=True)
    mm2 = lax.dot_general(xf, w2_ref[...], (((1,), (1,)), ((), ())),
                          preferred_element_type=jnp.float32)
    dist = jnp.sqrt(jnp.maximum((a2 + b2_ref[...]) - mm2, 0.0))
    mv = jnp.min(dist, axis=1, keepdims=True)
    ks = lax.broadcasted_iota(jnp.int32, dist.shape, 1)
    idx = jnp.min(jnp.where(dist == mv, ks, _K), axis=1)
    idx_ref[...] = idx[None, None, :].astype(jnp.int32)


def _tc_indices(xf, W2, b2):
    n = xf.shape[0]
    nb = n // _M
    return pl.pallas_call(
        _dist_argmin_body,
        grid=(nb,),
        in_specs=[
            pl.BlockSpec((_M, _EMB), lambda i: (i, 0)),
            pl.BlockSpec((_K, _EMB), lambda i: (0, 0)),
            pl.BlockSpec((1, _K), lambda i: (0, 0)),
        ],
        out_specs=pl.BlockSpec((1, 1, _M), lambda i: (i, 0, 0)),
        out_shape=jax.ShapeDtypeStruct((nb, 1, _M), jnp.int32),
        compiler_params=pltpu.CompilerParams(
            dimension_semantics=("arbitrary",)),
    )(xf, W2, b2)


_NC, _NS = 2, 16
_NW = _NC * _NS
_CH = 128


def _make_sc_gather(n):
    _BPW = n // _NW
    _NCHUNK = _BPW // _CH
    @functools.partial(
        pl.kernel,
        mesh=plsc.VectorSubcoreMesh(core_axis_name="c", subcore_axis_name="s"),
        out_type=jax.ShapeDtypeStruct((n, _EMB), jnp.float32),
        scratch_types=[
            pltpu.VMEM((_CH,), jnp.int32),
            pltpu.VMEM((_CH,), jnp.int32),
            pltpu.VMEM((_CH, _EMB), jnp.float32),
            pltpu.VMEM((_CH, _EMB), jnp.float32),
            pltpu.SemaphoreType.DMA,
            pltpu.SemaphoreType.DMA,
            pltpu.SemaphoreType.DMA,
            pltpu.SemaphoreType.DMA,
        ],
    )
    def _sc_gather(table_hbm, idx_hbm, out_hbm, idx_v0, idx_v1, rows_v0,
                   rows_v1, sg0, sg1, sw0, sw1):
        wid = lax.axis_index("s") * _NC + lax.axis_index("c")
        base = wid * _BPW
        idx_v = (idx_v0, idx_v1)
        rows_v = (rows_v0, rows_v1)
        sg = (sg0, sg1)
        sw = (sw0, sw1)
        gathers = [None] * _NCHUNK
        writes = [None] * _NCHUNK
        pltpu.sync_copy(idx_hbm.at[pl.ds(base, _CH)], idx_v0)
        gathers[0] = pltpu.async_copy(table_hbm.at[idx_v0], rows_v0, sg0)
        for ci in range(_NCHUNK):
            p = ci % 2
            if ci + 1 < _NCHUNK:
                q = (ci + 1) % 2
                pltpu.sync_copy(
                    idx_hbm.at[pl.ds(base + (ci + 1) * _CH, _CH)], idx_v[q])
                if ci >= 1:
                    writes[ci - 1].wait()
                gathers[ci + 1] = pltpu.async_copy(
                    table_hbm.at[idx_v[q]], rows_v[q], sg[q])
            gathers[ci].wait()
            writes[ci] = pltpu.async_copy(
                rows_v[p], out_hbm.at[pl.ds(base + ci * _CH, _CH)], sw[p])
        writes[_NCHUNK - 2].wait()
        writes[_NCHUNK - 1].wait()

    return _sc_gather


def kernel(x, W):
    b, c, h, w = x.shape
    n = b * h * w
    xf = jnp.transpose(x, (0, 2, 3, 1)).reshape(n, c)
    b2 = jnp.sum(W * W, axis=1)[None, :]
    idx = _tc_indices(xf, 2.0 * W, b2).reshape(n)
    qf = _make_sc_gather(n)(W, idx)
    quantized = jnp.transpose(qf.reshape(b, h, w, c), (0, 3, 1, 2))
    return (quantized, idx)


# speedup vs baseline: 1.5237x; 1.0673x over previous
import functools

import jax
import jax.numpy as jnp
from jax import lax
from jax.experimental import pallas as pl
from jax.experimental.pallas import tpu as pltpu
from jax.experimental.pallas import tpu_sc as plsc

_EMB = 256
_K = 1024
_M = 4096


def _dist_argmin_body(xf_ref, w2_ref, b2_ref, idx_ref):
    xf = xf_ref[...]
    a2 = jnp.sum(xf * xf, axis=1, keepdims=True)
    mm2 = lax.dot_general(xf, w2_ref[...], (((1,), (1,)), ((), ())),
                          preferred_element_type=jnp.float32)
    dist = jnp.sqrt(jnp.maximum((a2 + b2_ref[...]) - mm2, 0.0))
    mv = jnp.min(dist, axis=1, keepdims=True)
    ks = lax.broadcasted_iota(jnp.int32, dist.shape, 1)
    idx = jnp.min(jnp.where(dist == mv, ks, _K), axis=1)
    idx_ref[...] = idx[None, None, :].astype(jnp.int32)


def _tc_indices(xf, W2, b2):
    n = xf.shape[0]
    nb = n // _M
    return pl.pallas_call(
        _dist_argmin_body,
        grid=(nb,),
        in_specs=[
            pl.BlockSpec((_M, _EMB), lambda i: (i, 0)),
            pl.BlockSpec((_K, _EMB), lambda i: (0, 0)),
            pl.BlockSpec((1, _K), lambda i: (0, 0)),
        ],
        out_specs=pl.BlockSpec((1, 1, _M), lambda i: (i, 0, 0)),
        out_shape=jax.ShapeDtypeStruct((nb, 1, _M), jnp.int32),
        compiler_params=pltpu.CompilerParams(
            dimension_semantics=("arbitrary",)),
    )(xf, W2, b2)


_NC, _NS = 2, 16
_NW = _NC * _NS
_CH = 128


def _make_sc_gather(n):
    _BPW = n // _NW
    _NCHUNK = _BPW // _CH
    @functools.partial(
        pl.kernel,
        mesh=plsc.VectorSubcoreMesh(core_axis_name="c", subcore_axis_name="s"),
        out_type=jax.ShapeDtypeStruct((n, _EMB), jnp.float32),
        scratch_types=[
            pltpu.VMEM((_CH,), jnp.int32),
            pltpu.VMEM((_CH,), jnp.int32),
            pltpu.VMEM((_CH, _EMB), jnp.float32),
            pltpu.VMEM((_CH, _EMB), jnp.float32),
            pltpu.SemaphoreType.DMA,
            pltpu.SemaphoreType.DMA,
            pltpu.SemaphoreType.DMA,
            pltpu.SemaphoreType.DMA,
        ],
    )
    def _sc_gather(table_hbm, idx_hbm, out_hbm, idx_v0, idx_v1, rows_v0,
                   rows_v1, sg0, sg1, sw0, sw1):
        wid = lax.axis_index("s") * _NC + lax.axis_index("c")
        base = wid * _BPW
        idx_v = (idx_v0, idx_v1)
        rows_v = (rows_v0, rows_v1)
        sg = (sg0, sg1)
        sw = (sw0, sw1)
        gathers = [None] * _NCHUNK
        writes = [None] * _NCHUNK
        pltpu.sync_copy(idx_hbm.at[pl.ds(base, _CH)], idx_v0)
        gathers[0] = pltpu.async_copy(table_hbm.at[idx_v0], rows_v0, sg0)
        for ci in range(_NCHUNK):
            p = ci % 2
            if ci + 1 < _NCHUNK:
                q = (ci + 1) % 2
                pltpu.sync_copy(
                    idx_hbm.at[pl.ds(base + (ci + 1) * _CH, _CH)], idx_v[q])
                if ci >= 1:
                    writes[ci - 1].wait()
                gathers[ci + 1] = pltpu.async_copy(
                    table_hbm.at[idx_v[q]], rows_v[q], sg[q])
            gathers[ci].wait()
            writes[ci] = pltpu.async_copy(
                rows_v[p], out_hbm.at[pl.ds(base + ci * _CH, _CH)], sw[p])
        writes[_NCHUNK - 2].wait()
        writes[_NCHUNK - 1].wait()

    return _sc_gather


def kernel(x, W):
    b, c, h, w = x.shape
    n = b * h * w
    xf = jnp.transpose(x, (0, 2, 3, 1)).reshape(n, c)
    b2 = jnp.sum(W * W, axis=1)[None, :]
    idx = _tc_indices(xf, 2.0 * W, b2).reshape(n)
    qf = _make_sc_gather(n)(W, idx)
    quantized = jnp.transpose(qf.reshape(b, h, w, c), (0, 3, 1, 2))
    return (quantized, idx)
